# baseline (speedup 1.0000x reference)
import jax
import jax.numpy as jnp
from jax import lax
from jax.experimental import pallas as pl
from jax.experimental.pallas import tpu as pltpu
from jax.experimental.pallas import tpu_sc as plsc

VOCAB = 1000000
EMB_DIM = 64
BATCH = 16384
HIST = 50

NUM_CORES = 2
NUM_SUBCORES = 16
NUM_WORKERS = NUM_CORES * NUM_SUBCORES

S_PER_W = BATCH // NUM_WORKERS
S_CHUNK = 16
N_CHUNKS = S_PER_W // S_CHUNK


def _body(idx_hbm, table_hbm, out_hbm, idx_v, rows_v, gsem):
    wid = lax.axis_index("s") * NUM_CORES + lax.axis_index("c")
    base_s = wid * S_PER_W

    def chunk_body(i, carry):
        s0 = base_s + i * S_CHUNK
        pltpu.sync_copy(idx_hbm.at[pl.ds(s0, S_CHUNK)], idx_v)
        copies = [
            pltpu.async_copy(
                table_hbm.at[idx_v.at[j]],
                rows_v.at[j],
                gsem,
            )
            for j in range(S_CHUNK)
        ]
        for cp in copies:
            cp.wait()
        pltpu.sync_copy(rows_v, out_hbm.at[pl.ds(s0, S_CHUNK)])
        return carry

    lax.fori_loop(0, N_CHUNKS, chunk_body, 0)


@jax.jit
def kernel(batch, table):
    mesh = plsc.VectorSubcoreMesh(core_axis_name="c", subcore_axis_name="s")
    return pl.kernel(
        _body,
        out_type=jax.ShapeDtypeStruct((BATCH, HIST, EMB_DIM), jnp.float32),
        mesh=mesh,
        compiler_params=pltpu.CompilerParams(use_tc_tiling_on_sc=False),
        scratch_types=[
            pltpu.VMEM((S_CHUNK, HIST), jnp.int32),
            pltpu.VMEM((S_CHUNK, HIST, EMB_DIM), jnp.float32),
            pltpu.SemaphoreType.DMA,
        ],
    )(batch.astype(jnp.int32), table)
